# Initial kernel scaffold; baseline (speedup 1.0000x reference)
import jax
import jax.numpy as jnp
from jax.experimental import pallas as pl


def kernel(x, edge_index, edge_values, weight, bias):
    raise NotImplementedError("write your pallas kernel here")


# speedup vs baseline: 3.4403x; 3.4403x over previous
import functools

import jax
import jax.numpy as jnp
from jax import lax
from jax.experimental import pallas as pl
from jax.experimental.pallas import tpu as pltpu
from jax.experimental.pallas import tpu_sc as plsc

NC = 2
NS = 16
L = 16
C = 128


def _spmm_body(n_rows, n_chunks, x_hbm, col_hbm, row_hbm, val_hbm, out_hbm,
               col_v, row_v, val_v, rows_v, zbuf, acc, sem):
    c = lax.axis_index("c")
    s = lax.axis_index("s")
    f = x_hbm.shape[1]
    nj = f // L
    zrows = zbuf.shape[0]

    stripe = -(-(n_rows // NS) // 8) * 8
    start = s * stripe
    nrows = jnp.minimum(stripe, n_rows - start)

    zero = jnp.zeros((L,), jnp.float32)

    def zb(i, _):
        for j in range(nj):
            zbuf[i, pl.ds(j * L, L)] = zero
        return 0

    lax.fori_loop(0, zrows, zb, 0)
    nbig = nrows // zrows

    def zcp(i, _):
        pltpu.sync_copy(zbuf, acc.at[pl.ds(start + i * zrows, zrows)])
        return 0

    lax.fori_loop(0, nbig, zcp, 0)
    tail0 = start + nbig * zrows
    ntail = (nrows - nbig * zrows) // 8

    def zcp8(i, _):
        pltpu.sync_copy(zbuf.at[pl.ds(0, 8)], acc.at[pl.ds(tail0 + i * 8, 8)])
        return 0

    lax.fori_loop(0, ntail, zcp8, 0)
    plsc.subcore_barrier()

    base = (c * NS + s) * (n_chunks * C)

    def chunk(t, _):
        off = base + t * C
        pltpu.sync_copy(col_hbm.at[pl.ds(off, C)], col_v)
        pltpu.sync_copy(row_hbm.at[pl.ds(off, C)], row_v)
        pltpu.sync_copy(val_hbm.at[pl.ds(off, C)], val_v)
        pltpu.async_copy(x_hbm.at[col_v], rows_v, sem).wait()

        def scale(g, _):
            vg = val_v[pl.ds(g * L, L)]
            for l in range(L):
                e = g * L + l
                vv = vg[l]
                for j in range(nj):
                    rows_v[e, pl.ds(j * L, L)] = rows_v[e, pl.ds(j * L, L)] * vv
            return 0

        lax.fori_loop(0, C // L, scale, 0)
        pltpu.sync_copy(rows_v, acc.at[row_v], add=True)
        return 0

    lax.fori_loop(0, n_chunks, chunk, 0)

    plsc.subcore_barrier()
    cn = c * n_rows

    def wcp(i, _):
        o = start + i * zrows
        pltpu.sync_copy(acc.at[pl.ds(o, zrows)], out_hbm.at[pl.ds(cn + o, zrows)])
        return 0

    lax.fori_loop(0, nbig, wcp, 0)

    def wcp8(i, _):
        o = tail0 + i * 8
        pltpu.sync_copy(acc.at[pl.ds(o, 8)], out_hbm.at[pl.ds(cn + o, 8)])
        return 0

    lax.fori_loop(0, ntail, wcp8, 0)


def _spmm_sc(xmat, col, row, val, n_chunks):
    n, f = xmat.shape
    mesh = plsc.VectorSubcoreMesh(core_axis_name="c", subcore_axis_name="s")
    kern = functools.partial(
        pl.kernel,
        mesh=mesh,
        out_type=jax.ShapeDtypeStruct((NC * n, f), jnp.float32),
        scratch_types=[
            pltpu.VMEM((C,), jnp.int32),
            pltpu.VMEM((C,), jnp.int32),
            pltpu.VMEM((C,), jnp.float32),
            pltpu.VMEM((C, f), jnp.float32),
            pltpu.VMEM((128, f), jnp.float32),
            pltpu.VMEM_SHARED((n, f), jnp.float32),
            pltpu.SemaphoreType.DMA,
        ],
    )(functools.partial(_spmm_body, n, n_chunks))
    return kern(xmat, col, row, val)


def _mix1_block(x0_ref, pa_ref, pb_ref, w_ref, b_ref, x1_ref, acc_ref):
    x1 = pa_ref[...] + pb_ref[...]
    x1_ref[...] = x1
    w = w_ref[...]
    w02 = w[:, 0, :] - w[:, 2, :]
    w1 = w[:, 1, :]
    acc_ref[...] = (jnp.dot(x0_ref[...], w02, preferred_element_type=jnp.float32)
                    + jnp.dot(x1, w1, preferred_element_type=jnp.float32)
                    + b_ref[...])


def _mix2_block(acc_ref, qa_ref, qb_ref, w_ref, out_ref):
    w2 = w_ref[...][:, 2, :]
    q = qa_ref[...] + qb_ref[...]
    out_ref[...] = acc_ref[...] + jnp.dot(2.0 * q, w2,
                                          preferred_element_type=jnp.float32)


def kernel(x, edge_index, edge_values, weight, bias):
    b, v, fin = x.shape
    fin2, kk, fout = weight.shape
    n = b * v
    x0 = x.reshape(n, fin)

    row = edge_index[0].astype(jnp.int32)
    col = edge_index[1].astype(jnp.int32)
    e = row.shape[0]
    per_sub = NC * NS * C
    n_chunks = -(-e // per_sub)
    e_pad = n_chunks * per_sub
    pad = e_pad - e
    row = jnp.pad(row, (0, pad))
    col = jnp.pad(col, (0, pad))
    val = jnp.pad(edge_values, (0, pad))

    p = _spmm_sc(x0, col, row, val, n_chunks)

    rb = 1000
    nb = n // rb
    grid = (nb,)
    bias2 = bias.reshape(1, fout)
    x1, acc = pl.pallas_call(
        _mix1_block,
        grid=grid,
        in_specs=[
            pl.BlockSpec((rb, fin), lambda i: (i, 0)),
            pl.BlockSpec((rb, fin), lambda i: (i, 0)),
            pl.BlockSpec((rb, fin), lambda i: (i + nb, 0)),
            pl.BlockSpec((fin, kk, fout), lambda i: (0, 0, 0)),
            pl.BlockSpec((1, fout), lambda i: (0, 0)),
        ],
        out_specs=[
            pl.BlockSpec((rb, fin), lambda i: (i, 0)),
            pl.BlockSpec((rb, fout), lambda i: (i, 0)),
        ],
        out_shape=[
            jax.ShapeDtypeStruct((n, fin), jnp.float32),
            jax.ShapeDtypeStruct((n, fout), jnp.float32),
        ],
    )(x0, p, p, weight, bias2)

    q = _spmm_sc(x1, col, row, val, n_chunks)

    out = pl.pallas_call(
        _mix2_block,
        grid=grid,
        in_specs=[
            pl.BlockSpec((rb, fout), lambda i: (i, 0)),
            pl.BlockSpec((rb, fin), lambda i: (i, 0)),
            pl.BlockSpec((rb, fin), lambda i: (i + nb, 0)),
            pl.BlockSpec((fin, kk, fout), lambda i: (0, 0, 0)),
        ],
        out_specs=pl.BlockSpec((rb, fout), lambda i: (i, 0)),
        out_shape=jax.ShapeDtypeStruct((n, fout), jnp.float32),
    )(acc, q, q, weight)

    return out.reshape(b, v, fout)


# speedup vs baseline: 3.5182x; 1.0227x over previous
import functools

import jax
import jax.numpy as jnp
from jax import lax
from jax.experimental import pallas as pl
from jax.experimental.pallas import tpu as pltpu
from jax.experimental.pallas import tpu_sc as plsc

NC = 2
NS = 16
L = 16
C = 128


def _spmm_body(n_rows, n_chunks, x_hbm, col_hbm, row_hbm, val_hbm, out_hbm,
               col_v, row_v, val_v, rows_v, acc, sem_i, sem_g, sem_s):
    c = lax.axis_index("c")
    s = lax.axis_index("s")
    f = x_hbm.shape[1]
    nj = f // L
    zrows = rows_v.shape[1]

    stripe = -(-(n_rows // NS) // 8) * 8
    start = s * stripe
    nrows = jnp.minimum(stripe, n_rows - start)

    zero = jnp.zeros((L,), jnp.float32)

    def zb(i, _):
        for j in range(nj):
            rows_v[0, i, pl.ds(j * L, L)] = zero
        return 0

    lax.fori_loop(0, zrows, zb, 0)
    nbig = nrows // zrows

    def zcp(i, _):
        pltpu.sync_copy(rows_v.at[0], acc.at[pl.ds(start + i * zrows, zrows)])
        return 0

    lax.fori_loop(0, nbig, zcp, 0)
    tail0 = start + nbig * zrows
    ntail = (nrows - nbig * zrows) // 8

    def zcp8(i, _):
        pltpu.sync_copy(rows_v.at[0, pl.ds(0, 8)], acc.at[pl.ds(tail0 + i * 8, 8)])
        return 0

    lax.fori_loop(0, ntail, zcp8, 0)
    plsc.subcore_barrier()

    base = (c * NS + s) * (n_chunks * C)
    nt = n_chunks

    def start_idx(t):
        b = lax.rem(t, 3)
        off = base + t * C
        pltpu.make_async_copy(col_hbm.at[pl.ds(off, C)], col_v.at[b], sem_i.at[b]).start()
        pltpu.make_async_copy(row_hbm.at[pl.ds(off, C)], row_v.at[b], sem_i.at[b]).start()
        pltpu.make_async_copy(val_hbm.at[pl.ds(off, C)], val_v.at[b], sem_i.at[b]).start()

    def wait_idx(t):
        b = lax.rem(t, 3)
        pltpu.make_async_copy(col_hbm.at[pl.ds(base, C)], col_v.at[b], sem_i.at[b]).wait()
        pltpu.make_async_copy(row_hbm.at[pl.ds(base, C)], row_v.at[b], sem_i.at[b]).wait()
        pltpu.make_async_copy(val_hbm.at[pl.ds(base, C)], val_v.at[b], sem_i.at[b]).wait()

    def start_gather(t):
        b = lax.rem(t, 3)
        pltpu.make_async_copy(x_hbm.at[col_v.at[b]], rows_v.at[b], sem_g.at[b]).start()

    def wait_gather(t):
        b = lax.rem(t, 3)
        pltpu.make_async_copy(x_hbm.at[col_v.at[b]], rows_v.at[b], sem_g.at[b]).wait()

    def start_scatter(t):
        b = lax.rem(t, 3)
        pltpu.async_copy(rows_v.at[b], acc.at[row_v.at[b]], sem_s.at[b], add=True)

    def wait_scatter(t):
        b = lax.rem(t, 3)
        pltpu.make_async_copy(rows_v.at[b], acc.at[row_v.at[b]], sem_s.at[b]).wait()

    start_idx(0)
    start_idx(1)
    wait_idx(0)
    start_gather(0)

    def chunk(t, _):
        b = lax.rem(t, 3)
        wait_gather(t)

        @pl.when(t + 1 < nt)
        def _():
            wait_idx(t + 1)
            start_gather(t + 1)

        def scale(g, _):
            vg = val_v[b, pl.ds(g * L, L)]
            for l in range(L):
                vv = vg[l]
                for j in range(nj):
                    rows_v[b, g * L + l, pl.ds(j * L, L)] = (
                        rows_v[b, g * L + l, pl.ds(j * L, L)] * vv)
            return 0

        lax.fori_loop(0, C // L, scale, 0)

        @pl.when(t >= 1)
        def _():
            wait_scatter(t - 1)

        start_scatter(t)

        @pl.when(t + 2 < nt)
        def _():
            start_idx(t + 2)

        return 0

    lax.fori_loop(0, nt, chunk, 0)
    wait_scatter(nt - 1)

    plsc.subcore_barrier()
    cn = c * n_rows

    def wcp(i, _):
        o = start + i * zrows
        pltpu.sync_copy(acc.at[pl.ds(o, zrows)], out_hbm.at[pl.ds(cn + o, zrows)])
        return 0

    lax.fori_loop(0, nbig, wcp, 0)

    def wcp8(i, _):
        o = tail0 + i * 8
        pltpu.sync_copy(acc.at[pl.ds(o, 8)], out_hbm.at[pl.ds(cn + o, 8)])
        return 0

    lax.fori_loop(0, ntail, wcp8, 0)


def _spmm_sc(xmat, col, row, val, n_chunks):
    n, f = xmat.shape
    mesh = plsc.VectorSubcoreMesh(core_axis_name="c", subcore_axis_name="s")
    kern = functools.partial(
        pl.kernel,
        mesh=mesh,
        out_type=jax.ShapeDtypeStruct((NC * n, f), jnp.float32),
        scratch_types=[
            pltpu.VMEM((3, C), jnp.int32),
            pltpu.VMEM((3, C), jnp.int32),
            pltpu.VMEM((3, C), jnp.float32),
            pltpu.VMEM((3, C, f), jnp.float32),
            pltpu.VMEM_SHARED((n, f), jnp.float32),
            pltpu.SemaphoreType.DMA((3,)),
            pltpu.SemaphoreType.DMA((3,)),
            pltpu.SemaphoreType.DMA((3,)),
        ],
    )(functools.partial(_spmm_body, n, n_chunks))
    return kern(xmat, col, row, val)


def _mix1_block(x0_ref, pa_ref, pb_ref, w_ref, b_ref, x1_ref, acc_ref):
    x1 = pa_ref[...] + pb_ref[...]
    x1_ref[...] = x1
    w = w_ref[...]
    w02 = w[:, 0, :] - w[:, 2, :]
    w1 = w[:, 1, :]
    acc_ref[...] = (jnp.dot(x0_ref[...], w02, preferred_element_type=jnp.float32)
                    + jnp.dot(x1, w1, preferred_element_type=jnp.float32)
                    + b_ref[...])


def _mix2_block(acc_ref, qa_ref, qb_ref, w_ref, out_ref):
    w2 = w_ref[...][:, 2, :]
    q = qa_ref[...] + qb_ref[...]
    out_ref[...] = acc_ref[...] + jnp.dot(2.0 * q, w2,
                                          preferred_element_type=jnp.float32)


def kernel(x, edge_index, edge_values, weight, bias):
    b, v, fin = x.shape
    fin2, kk, fout = weight.shape
    n = b * v
    x0 = x.reshape(n, fin)

    row = edge_index[0].astype(jnp.int32)
    col = edge_index[1].astype(jnp.int32)
    e = row.shape[0]
    per_sub = NC * NS * C
    n_chunks = -(-e // per_sub)
    e_pad = n_chunks * per_sub
    pad = e_pad - e
    row = jnp.pad(row, (0, pad))
    col = jnp.pad(col, (0, pad))
    val = jnp.pad(edge_values, (0, pad))

    p = _spmm_sc(x0, col, row, val, n_chunks)

    rb = 1000
    nb = n // rb
    grid = (nb,)
    bias2 = bias.reshape(1, fout)
    x1, acc = pl.pallas_call(
        _mix1_block,
        grid=grid,
        in_specs=[
            pl.BlockSpec((rb, fin), lambda i: (i, 0)),
            pl.BlockSpec((rb, fin), lambda i: (i, 0)),
            pl.BlockSpec((rb, fin), lambda i: (i + nb, 0)),
            pl.BlockSpec((fin, kk, fout), lambda i: (0, 0, 0)),
            pl.BlockSpec((1, fout), lambda i: (0, 0)),
        ],
        out_specs=[
            pl.BlockSpec((rb, fin), lambda i: (i, 0)),
            pl.BlockSpec((rb, fout), lambda i: (i, 0)),
        ],
        out_shape=[
            jax.ShapeDtypeStruct((n, fin), jnp.float32),
            jax.ShapeDtypeStruct((n, fout), jnp.float32),
        ],
    )(x0, p, p, weight, bias2)

    q = _spmm_sc(x1, col, row, val, n_chunks)

    out = pl.pallas_call(
        _mix2_block,
        grid=grid,
        in_specs=[
            pl.BlockSpec((rb, fout), lambda i: (i, 0)),
            pl.BlockSpec((rb, fin), lambda i: (i, 0)),
            pl.BlockSpec((rb, fin), lambda i: (i + nb, 0)),
            pl.BlockSpec((fin, kk, fout), lambda i: (0, 0, 0)),
        ],
        out_specs=pl.BlockSpec((rb, fout), lambda i: (i, 0)),
        out_shape=jax.ShapeDtypeStruct((n, fout), jnp.float32),
    )(acc, q, q, weight)

    return out.reshape(b, v, fout)


# speedup vs baseline: 5.4261x; 1.5423x over previous
import functools

import jax
import jax.numpy as jnp
from jax import lax
from jax.experimental import pallas as pl
from jax.experimental.pallas import tpu as pltpu
from jax.experimental.pallas import tpu_sc as plsc

NC = 2
NS = 16
L = 16
C = 128


def _spmm_body(n_rows, n_chunks, x_hbm, col_hbm, row_hbm, val_hbm, out_hbm,
               col_v, row_v, val_v, rows_v, acc, sem_i, sem_g, sem_s):
    c = lax.axis_index("c")
    s = lax.axis_index("s")
    f = x_hbm.shape[1]
    nj = f // L
    zrows = rows_v.shape[1]

    stripe = -(-(n_rows // NS) // 8) * 8
    start = s * stripe
    nrows = jnp.minimum(stripe, n_rows - start)

    zero = jnp.zeros((L,), jnp.float32)

    def zb(i, _):
        for j in range(nj):
            rows_v[0, i, pl.ds(j * L, L)] = zero
        return 0

    lax.fori_loop(0, zrows, zb, 0)
    nbig = nrows // zrows

    def zcp(i, _):
        pltpu.sync_copy(rows_v.at[0], acc.at[pl.ds(start + i * zrows, zrows)])
        return 0

    lax.fori_loop(0, nbig, zcp, 0)
    tail0 = start + nbig * zrows
    ntail = (nrows - nbig * zrows) // 8

    def zcp8(i, _):
        pltpu.sync_copy(rows_v.at[0, pl.ds(0, 8)], acc.at[pl.ds(tail0 + i * 8, 8)])
        return 0

    lax.fori_loop(0, ntail, zcp8, 0)
    plsc.subcore_barrier()

    base = (c * NS + s) * (n_chunks * C)
    nt = n_chunks

    def start_idx(t):
        b = lax.rem(t, 3)
        off = base + t * C
        pltpu.make_async_copy(col_hbm.at[pl.ds(off, C)], col_v.at[b], sem_i.at[b]).start()
        pltpu.make_async_copy(row_hbm.at[pl.ds(off, C)], row_v.at[b], sem_i.at[b]).start()
        pltpu.make_async_copy(val_hbm.at[pl.ds(off, C)], val_v.at[b], sem_i.at[b]).start()

    def wait_idx(t):
        b = lax.rem(t, 3)
        pltpu.make_async_copy(col_hbm.at[pl.ds(base, C)], col_v.at[b], sem_i.at[b]).wait()
        pltpu.make_async_copy(row_hbm.at[pl.ds(base, C)], row_v.at[b], sem_i.at[b]).wait()
        pltpu.make_async_copy(val_hbm.at[pl.ds(base, C)], val_v.at[b], sem_i.at[b]).wait()

    def start_gather(t):
        b = lax.rem(t, 3)
        pltpu.make_async_copy(x_hbm.at[col_v.at[b]], rows_v.at[b], sem_g.at[b]).start()

    def wait_gather(t):
        b = lax.rem(t, 3)
        pltpu.make_async_copy(x_hbm.at[col_v.at[b]], rows_v.at[b], sem_g.at[b]).wait()

    def start_scatter(t):
        b = lax.rem(t, 3)
        pltpu.async_copy(rows_v.at[b], acc.at[row_v.at[b]], sem_s.at[b], add=True)

    def wait_scatter(t):
        b = lax.rem(t, 3)
        pltpu.make_async_copy(rows_v.at[b], acc.at[row_v.at[b]], sem_s.at[b]).wait()

    start_idx(0)
    start_idx(1)
    wait_idx(0)
    start_gather(0)

    def chunk(t, _):
        b = lax.rem(t, 3)
        wait_gather(t)

        @pl.when(t + 1 < nt)
        def _():
            wait_idx(t + 1)
            start_gather(t + 1)

        @plsc.parallel_loop(0, C // L, 1, unroll=2)
        def scale(g):
            vg = val_v[b, pl.ds(g * L, L)]
            for l in range(L):
                e = g * L + l
                vv = vg[l]
                segs = [rows_v[b, e, pl.ds(j * L, L)] * vv for j in range(nj)]
                for j in range(nj):
                    rows_v[b, e, pl.ds(j * L, L)] = segs[j]

        @pl.when(t >= 1)
        def _():
            wait_scatter(t - 1)

        start_scatter(t)

        @pl.when(t + 2 < nt)
        def _():
            start_idx(t + 2)

        return 0

    lax.fori_loop(0, nt, chunk, 0)
    wait_scatter(nt - 1)

    plsc.subcore_barrier()
    cn = c * n_rows

    def wcp(i, _):
        o = start + i * zrows
        pltpu.sync_copy(acc.at[pl.ds(o, zrows)], out_hbm.at[pl.ds(cn + o, zrows)])
        return 0

    lax.fori_loop(0, nbig, wcp, 0)

    def wcp8(i, _):
        o = tail0 + i * 8
        pltpu.sync_copy(acc.at[pl.ds(o, 8)], out_hbm.at[pl.ds(cn + o, 8)])
        return 0

    lax.fori_loop(0, ntail, wcp8, 0)


def _spmm_sc(xmat, col, row, val, n_chunks):
    n, f = xmat.shape
    mesh = plsc.VectorSubcoreMesh(core_axis_name="c", subcore_axis_name="s")
    kern = functools.partial(
        pl.kernel,
        mesh=mesh,
        out_type=jax.ShapeDtypeStruct((NC * n, f), jnp.float32),
        scratch_types=[
            pltpu.VMEM((3, C), jnp.int32),
            pltpu.VMEM((3, C), jnp.int32),
            pltpu.VMEM((3, C), jnp.float32),
            pltpu.VMEM((3, C, f), jnp.float32),
            pltpu.VMEM_SHARED((n, f), jnp.float32),
            pltpu.SemaphoreType.DMA((3,)),
            pltpu.SemaphoreType.DMA((3,)),
            pltpu.SemaphoreType.DMA((3,)),
        ],
    )(functools.partial(_spmm_body, n, n_chunks))
    return kern(xmat, col, row, val)


def _mix1_block(x0_ref, pa_ref, pb_ref, w_ref, b_ref, x1_ref, acc_ref):
    x1 = pa_ref[...] + pb_ref[...]
    x1_ref[...] = x1
    w = w_ref[...]
    w02 = w[:, 0, :] - w[:, 2, :]
    w1 = w[:, 1, :]
    acc_ref[...] = (jnp.dot(x0_ref[...], w02, preferred_element_type=jnp.float32)
                    + jnp.dot(x1, w1, preferred_element_type=jnp.float32)
                    + b_ref[...])


def _mix2_block(acc_ref, qa_ref, qb_ref, w_ref, out_ref):
    w2 = w_ref[...][:, 2, :]
    q = qa_ref[...] + qb_ref[...]
    out_ref[...] = acc_ref[...] + jnp.dot(2.0 * q, w2,
                                          preferred_element_type=jnp.float32)


def kernel(x, edge_index, edge_values, weight, bias):
    b, v, fin = x.shape
    fin2, kk, fout = weight.shape
    n = b * v
    x0 = x.reshape(n, fin)

    row = edge_index[0].astype(jnp.int32)
    col = edge_index[1].astype(jnp.int32)
    e = row.shape[0]
    per_sub = NC * NS * C
    n_chunks = -(-e // per_sub)
    e_pad = n_chunks * per_sub
    pad = e_pad - e
    row = jnp.pad(row, (0, pad))
    col = jnp.pad(col, (0, pad))
    val = jnp.pad(edge_values, (0, pad))

    p = _spmm_sc(x0, col, row, val, n_chunks)

    rb = 1000
    nb = n // rb
    grid = (nb,)
    bias2 = bias.reshape(1, fout)
    x1, acc = pl.pallas_call(
        _mix1_block,
        grid=grid,
        in_specs=[
            pl.BlockSpec((rb, fin), lambda i: (i, 0)),
            pl.BlockSpec((rb, fin), lambda i: (i, 0)),
            pl.BlockSpec((rb, fin), lambda i: (i + nb, 0)),
            pl.BlockSpec((fin, kk, fout), lambda i: (0, 0, 0)),
            pl.BlockSpec((1, fout), lambda i: (0, 0)),
        ],
        out_specs=[
            pl.BlockSpec((rb, fin), lambda i: (i, 0)),
            pl.BlockSpec((rb, fout), lambda i: (i, 0)),
        ],
        out_shape=[
            jax.ShapeDtypeStruct((n, fin), jnp.float32),
            jax.ShapeDtypeStruct((n, fout), jnp.float32),
        ],
    )(x0, p, p, weight, bias2)

    q = _spmm_sc(x1, col, row, val, n_chunks)

    out = pl.pallas_call(
        _mix2_block,
        grid=grid,
        in_specs=[
            pl.BlockSpec((rb, fout), lambda i: (i, 0)),
            pl.BlockSpec((rb, fin), lambda i: (i, 0)),
            pl.BlockSpec((rb, fin), lambda i: (i + nb, 0)),
            pl.BlockSpec((fin, kk, fout), lambda i: (0, 0, 0)),
        ],
        out_specs=pl.BlockSpec((rb, fout), lambda i: (i, 0)),
        out_shape=jax.ShapeDtypeStruct((n, fout), jnp.float32),
    )(acc, q, q, weight)

    return out.reshape(b, v, fout)


# speedup vs baseline: 6.0381x; 1.1128x over previous
import functools

import jax
import jax.numpy as jnp
from jax import lax
from jax.experimental import pallas as pl
from jax.experimental.pallas import tpu as pltpu
from jax.experimental.pallas import tpu_sc as plsc

NC = 2
NS = 16
L = 16
C = 128


def _spmm_body(n_rows, t0, t1, x_hbm, col_hbm, row_hbm, val_hbm, out_hbm,
               col_v, row_v, val_v, rows_v, acc, sem_i, sem_g, sem_s):
    c = lax.axis_index("c")
    s = lax.axis_index("s")
    f = x_hbm.shape[1]
    nj = f // L
    zrows = rows_v.shape[1]

    stripe = -(-(n_rows // NS) // 8) * 8
    start = s * stripe
    nrows = jnp.minimum(stripe, n_rows - start)

    zero = jnp.zeros((L,), jnp.float32)

    def zb(i, _):
        for j in range(nj):
            rows_v[0, i, pl.ds(j * L, L)] = zero
        return 0

    lax.fori_loop(0, zrows, zb, 0)
    nbig = nrows // zrows

    def zcp(i, _):
        pltpu.sync_copy(rows_v.at[0], acc.at[pl.ds(start + i * zrows, zrows)])
        return 0

    lax.fori_loop(0, nbig, zcp, 0)
    tail0 = start + nbig * zrows
    ntail = (nrows - nbig * zrows) // 8

    def zcp8(i, _):
        pltpu.sync_copy(rows_v.at[0, pl.ds(0, 8)], acc.at[pl.ds(tail0 + i * 8, 8)])
        return 0

    lax.fori_loop(0, ntail, zcp8, 0)
    plsc.subcore_barrier()

    base = jnp.where(c == 0, s * t0, NS * t0 + s * t1) * C
    nt = jnp.where(c == 0, t0, t1)

    def start_idx(t):
        b = lax.rem(t, 3)
        off = base + t * C
        pltpu.make_async_copy(col_hbm.at[pl.ds(off, C)], col_v.at[b], sem_i.at[b]).start()
        pltpu.make_async_copy(row_hbm.at[pl.ds(off, C)], row_v.at[b], sem_i.at[b]).start()
        pltpu.make_async_copy(val_hbm.at[pl.ds(off, C)], val_v.at[b], sem_i.at[b]).start()

    def wait_idx(t):
        b = lax.rem(t, 3)
        pltpu.make_async_copy(col_hbm.at[pl.ds(base, C)], col_v.at[b], sem_i.at[b]).wait()
        pltpu.make_async_copy(row_hbm.at[pl.ds(base, C)], row_v.at[b], sem_i.at[b]).wait()
        pltpu.make_async_copy(val_hbm.at[pl.ds(base, C)], val_v.at[b], sem_i.at[b]).wait()

    def start_gather(t):
        b = lax.rem(t, 3)
        pltpu.make_async_copy(x_hbm.at[col_v.at[b]], rows_v.at[b], sem_g.at[b]).start()

    def wait_gather(t):
        b = lax.rem(t, 3)
        pltpu.make_async_copy(x_hbm.at[col_v.at[b]], rows_v.at[b], sem_g.at[b]).wait()

    def start_scatter(t):
        b = lax.rem(t, 3)
        pltpu.async_copy(rows_v.at[b], acc.at[row_v.at[b]], sem_s.at[b], add=True)

    def wait_scatter(t):
        b = lax.rem(t, 3)
        pltpu.make_async_copy(rows_v.at[b], acc.at[row_v.at[b]], sem_s.at[b]).wait()

    start_idx(0)
    start_idx(1)
    wait_idx(0)
    start_gather(0)

    def chunk(t, _):
        b = lax.rem(t, 3)
        wait_gather(t)

        @pl.when(t + 1 < nt)
        def _():
            wait_idx(t + 1)
            start_gather(t + 1)

        @plsc.parallel_loop(0, C // L, 1, unroll=2)
        def scale(g):
            vg = val_v[b, pl.ds(g * L, L)]
            for l in range(L):
                e = g * L + l
                vv = vg[l]
                segs = [rows_v[b, e, pl.ds(j * L, L)] * vv for j in range(nj)]
                for j in range(nj):
                    rows_v[b, e, pl.ds(j * L, L)] = segs[j]

        @pl.when(t >= 1)
        def _():
            wait_scatter(t - 1)

        start_scatter(t)

        @pl.when(t + 2 < nt)
        def _():
            start_idx(t + 2)

        return 0

    lax.fori_loop(0, nt, chunk, 0)
    wait_scatter(nt - 1)

    plsc.subcore_barrier()
    cn = c * n_rows

    def wcp(i, _):
        o = start + i * zrows
        pltpu.sync_copy(acc.at[pl.ds(o, zrows)], out_hbm.at[pl.ds(cn + o, zrows)])
        return 0

    lax.fori_loop(0, nbig, wcp, 0)

    def wcp8(i, _):
        o = tail0 + i * 8
        pltpu.sync_copy(acc.at[pl.ds(o, 8)], out_hbm.at[pl.ds(cn + o, 8)])
        return 0

    lax.fori_loop(0, ntail, wcp8, 0)


def _spmm_sc(xmat, col, row, val, t0, t1):
    n, f = xmat.shape
    mesh = plsc.VectorSubcoreMesh(core_axis_name="c", subcore_axis_name="s")
    kern = functools.partial(
        pl.kernel,
        mesh=mesh,
        out_type=jax.ShapeDtypeStruct((NC * n, f), jnp.float32),
        scratch_types=[
            pltpu.VMEM((3, C), jnp.int32),
            pltpu.VMEM((3, C), jnp.int32),
            pltpu.VMEM((3, C), jnp.float32),
            pltpu.VMEM((3, C, f), jnp.float32),
            pltpu.VMEM_SHARED((n, f), jnp.float32),
            pltpu.SemaphoreType.DMA((3,)),
            pltpu.SemaphoreType.DMA((3,)),
            pltpu.SemaphoreType.DMA((3,)),
        ],
    )(functools.partial(_spmm_body, n, t0, t1))
    return kern(xmat, col, row, val)


def _mix1_block(x0_ref, pa_ref, pb_ref, w_ref, b_ref, x1_ref, acc_ref):
    x1 = pa_ref[...] + pb_ref[...]
    x1_ref[...] = x1
    w = w_ref[...]
    w02 = w[:, 0, :] - w[:, 2, :]
    w1 = w[:, 1, :]
    acc_ref[...] = (jnp.dot(x0_ref[...], w02, preferred_element_type=jnp.float32)
                    + jnp.dot(x1, w1, preferred_element_type=jnp.float32)
                    + b_ref[...])


def _mix2_block(acc_ref, qa_ref, qb_ref, w_ref, out_ref):
    w2 = w_ref[...][:, 2, :]
    q = qa_ref[...] + qb_ref[...]
    out_ref[...] = acc_ref[...] + jnp.dot(2.0 * q, w2,
                                          preferred_element_type=jnp.float32)


def kernel(x, edge_index, edge_values, weight, bias):
    b, v, fin = x.shape
    fin2, kk, fout = weight.shape
    n = b * v
    x0 = x.reshape(n, fin)

    row = edge_index[0].astype(jnp.int32)
    col = edge_index[1].astype(jnp.int32)
    e = row.shape[0]
    per_sub = NC * NS * C
    total = NC * (-(-e // per_sub))
    t0 = (2 * total) // 3
    t1 = total - t0
    e_pad = NS * C * total
    pad = e_pad - e
    row = jnp.pad(row, (0, pad))
    col = jnp.pad(col, (0, pad))
    val = jnp.pad(edge_values, (0, pad))

    p = _spmm_sc(x0, col, row, val, t0, t1)

    rb = 1000
    nb = n // rb
    grid = (nb,)
    bias2 = bias.reshape(1, fout)
    x1, acc = pl.pallas_call(
        _mix1_block,
        grid=grid,
        in_specs=[
            pl.BlockSpec((rb, fin), lambda i: (i, 0)),
            pl.BlockSpec((rb, fin), lambda i: (i, 0)),
            pl.BlockSpec((rb, fin), lambda i: (i + nb, 0)),
            pl.BlockSpec((fin, kk, fout), lambda i: (0, 0, 0)),
            pl.BlockSpec((1, fout), lambda i: (0, 0)),
        ],
        out_specs=[
            pl.BlockSpec((rb, fin), lambda i: (i, 0)),
            pl.BlockSpec((rb, fout), lambda i: (i, 0)),
        ],
        out_shape=[
            jax.ShapeDtypeStruct((n, fin), jnp.float32),
            jax.ShapeDtypeStruct((n, fout), jnp.float32),
        ],
    )(x0, p, p, weight, bias2)

    q = _spmm_sc(x1, col, row, val, t0, t1)

    out = pl.pallas_call(
        _mix2_block,
        grid=grid,
        in_specs=[
            pl.BlockSpec((rb, fout), lambda i: (i, 0)),
            pl.BlockSpec((rb, fin), lambda i: (i, 0)),
            pl.BlockSpec((rb, fin), lambda i: (i + nb, 0)),
            pl.BlockSpec((fin, kk, fout), lambda i: (0, 0, 0)),
        ],
        out_specs=pl.BlockSpec((rb, fout), lambda i: (i, 0)),
        out_shape=jax.ShapeDtypeStruct((n, fout), jnp.float32),
    )(acc, q, q, weight)

    return out.reshape(b, v, fout)


# speedup vs baseline: 6.0952x; 1.0095x over previous
import functools

import jax
import jax.numpy as jnp
from jax import lax
from jax.experimental import pallas as pl
from jax.experimental.pallas import tpu as pltpu
from jax.experimental.pallas import tpu_sc as plsc

NC = 2
NS = 16
L = 16
C = 128


def _spmm_body(n_rows, t0, t1, x_hbm, col_hbm, row_hbm, val_hbm, out_hbm,
               col_v, row_v, val_v, rows_v, acc, sem_i, sem_g, sem_s):
    c = lax.axis_index("c")
    s = lax.axis_index("s")
    f = x_hbm.shape[1]
    nj = f // L
    zrows = rows_v.shape[1]

    stripe = -(-(n_rows // NS) // 8) * 8
    start = s * stripe
    nrows = jnp.minimum(stripe, n_rows - start)

    zero = jnp.zeros((L,), jnp.float32)

    def zb(i, _):
        for j in range(nj):
            rows_v[0, i, pl.ds(j * L, L)] = zero
        return 0

    nbig = nrows // zrows
    tail0 = start + nbig * zrows
    ntail = (nrows - nbig * zrows) // 8

    with jax.named_scope("zero_acc"):
        lax.fori_loop(0, zrows, zb, 0)

        def zcp(i, _):
            pltpu.sync_copy(rows_v.at[0], acc.at[pl.ds(start + i * zrows, zrows)])
            return 0

        lax.fori_loop(0, nbig, zcp, 0)

        def zcp8(i, _):
            pltpu.sync_copy(rows_v.at[0, pl.ds(0, 8)], acc.at[pl.ds(tail0 + i * 8, 8)])
            return 0

        lax.fori_loop(0, ntail, zcp8, 0)
        plsc.subcore_barrier()

    base = jnp.where(c == 0, s * t0, NS * t0 + s * t1) * C
    nt = jnp.where(c == 0, t0, t1)

    def start_idx(t):
        b = lax.rem(t, 3)
        off = base + t * C
        pltpu.make_async_copy(col_hbm.at[pl.ds(off, C)], col_v.at[b], sem_i.at[b]).start()
        pltpu.make_async_copy(row_hbm.at[pl.ds(off, C)], row_v.at[b], sem_i.at[b]).start()
        pltpu.make_async_copy(val_hbm.at[pl.ds(off, C)], val_v.at[b], sem_i.at[b]).start()

    def wait_idx(t):
        b = lax.rem(t, 3)
        pltpu.make_async_copy(col_hbm.at[pl.ds(base, C)], col_v.at[b], sem_i.at[b]).wait()
        pltpu.make_async_copy(row_hbm.at[pl.ds(base, C)], row_v.at[b], sem_i.at[b]).wait()
        pltpu.make_async_copy(val_hbm.at[pl.ds(base, C)], val_v.at[b], sem_i.at[b]).wait()

    def start_gather(t):
        b = lax.rem(t, 3)
        pltpu.make_async_copy(x_hbm.at[col_v.at[b]], rows_v.at[b], sem_g.at[b]).start()

    def wait_gather(t):
        b = lax.rem(t, 3)
        pltpu.make_async_copy(x_hbm.at[col_v.at[b]], rows_v.at[b], sem_g.at[b]).wait()

    def start_scatter(t):
        b = lax.rem(t, 3)
        pltpu.async_copy(rows_v.at[b], acc.at[row_v.at[b]], sem_s.at[b], add=True)

    def wait_scatter(t):
        b = lax.rem(t, 3)
        pltpu.make_async_copy(rows_v.at[b], acc.at[row_v.at[b]], sem_s.at[b]).wait()

    sco = jax.named_scope("edge_loop")
    sco.__enter__()
    start_idx(0)
    start_idx(1)
    wait_idx(0)
    start_gather(0)

    def chunk(t, _):
        b = lax.rem(t, 3)
        wait_gather(t)

        @pl.when(t + 1 < nt)
        def _():
            wait_idx(t + 1)
            start_gather(t + 1)

        @plsc.parallel_loop(0, C // L, 1, unroll=2)
        def scale(g):
            vg = val_v[b, pl.ds(g * L, L)]
            for l in range(L):
                e = g * L + l
                vv = vg[l]
                segs = [rows_v[b, e, pl.ds(j * L, L)] * vv for j in range(nj)]
                for j in range(nj):
                    rows_v[b, e, pl.ds(j * L, L)] = segs[j]

        @pl.when(t >= 1)
        def _():
            wait_scatter(t - 1)

        start_scatter(t)

        @pl.when(t + 2 < nt)
        def _():
            start_idx(t + 2)

        return 0

    lax.fori_loop(0, nt, chunk, 0)
    wait_scatter(nt - 1)
    sco.__exit__(None, None, None)

    with jax.named_scope("writeout"):
        plsc.subcore_barrier()
        cn = c * n_rows

        def wcp(i, _):
            o = start + i * zrows
            pltpu.sync_copy(acc.at[pl.ds(o, zrows)], out_hbm.at[pl.ds(cn + o, zrows)])
            return 0

        lax.fori_loop(0, nbig, wcp, 0)

        def wcp8(i, _):
            o = tail0 + i * 8
            pltpu.sync_copy(acc.at[pl.ds(o, 8)], out_hbm.at[pl.ds(cn + o, 8)])
            return 0

        lax.fori_loop(0, ntail, wcp8, 0)


def _spmm_sc(xmat, col, row, val, t0, t1):
    n, f = xmat.shape
    mesh = plsc.VectorSubcoreMesh(core_axis_name="c", subcore_axis_name="s")
    kern = functools.partial(
        pl.kernel,
        mesh=mesh,
        out_type=jax.ShapeDtypeStruct((NC * n, f), jnp.float32),
        scratch_types=[
            pltpu.VMEM((3, C), jnp.int32),
            pltpu.VMEM((3, C), jnp.int32),
            pltpu.VMEM((3, C), jnp.float32),
            pltpu.VMEM((3, C, f), jnp.float32),
            pltpu.VMEM_SHARED((n, f), jnp.float32),
            pltpu.SemaphoreType.DMA((3,)),
            pltpu.SemaphoreType.DMA((3,)),
            pltpu.SemaphoreType.DMA((3,)),
        ],
    )(functools.partial(_spmm_body, n, t0, t1))
    return kern(xmat, col, row, val)


def _mix1_block(x0_ref, pa_ref, pb_ref, w_ref, b_ref, x1_ref, acc_ref):
    x1 = pa_ref[...] + pb_ref[...]
    x1_ref[...] = x1
    w = w_ref[...]
    w02 = w[:, 0, :] - w[:, 2, :]
    w1 = w[:, 1, :]
    acc_ref[...] = (jnp.dot(x0_ref[...], w02, preferred_element_type=jnp.float32)
                    + jnp.dot(x1, w1, preferred_element_type=jnp.float32)
                    + b_ref[...])


def _mix2_block(acc_ref, qa_ref, qb_ref, w_ref, out_ref):
    w2 = w_ref[...][:, 2, :]
    q = qa_ref[...] + qb_ref[...]
    out_ref[...] = acc_ref[...] + jnp.dot(2.0 * q, w2,
                                          preferred_element_type=jnp.float32)


def kernel(x, edge_index, edge_values, weight, bias):
    b, v, fin = x.shape
    fin2, kk, fout = weight.shape
    n = b * v
    x0 = x.reshape(n, fin)

    row = edge_index[0].astype(jnp.int32)
    col = edge_index[1].astype(jnp.int32)
    e = row.shape[0]
    per_sub = NC * NS * C
    total = NC * (-(-e // per_sub))
    t0 = (2 * total) // 3
    t1 = total - t0
    e_pad = NS * C * total
    pad = e_pad - e
    row = jnp.pad(row, (0, pad))
    col = jnp.pad(col, (0, pad))
    val = jnp.pad(edge_values, (0, pad))

    p = _spmm_sc(x0, col, row, val, t0, t1)

    rb = 1000
    nb = n // rb
    grid = (nb,)
    bias2 = bias.reshape(1, fout)
    x1, acc = pl.pallas_call(
        _mix1_block,
        grid=grid,
        in_specs=[
            pl.BlockSpec((rb, fin), lambda i: (i, 0)),
            pl.BlockSpec((rb, fin), lambda i: (i, 0)),
            pl.BlockSpec((rb, fin), lambda i: (i + nb, 0)),
            pl.BlockSpec((fin, kk, fout), lambda i: (0, 0, 0)),
            pl.BlockSpec((1, fout), lambda i: (0, 0)),
        ],
        out_specs=[
            pl.BlockSpec((rb, fin), lambda i: (i, 0)),
            pl.BlockSpec((rb, fout), lambda i: (i, 0)),
        ],
        out_shape=[
            jax.ShapeDtypeStruct((n, fin), jnp.float32),
            jax.ShapeDtypeStruct((n, fout), jnp.float32),
        ],
    )(x0, p, p, weight, bias2)

    q = _spmm_sc(x1, col, row, val, t0, t1)

    out = pl.pallas_call(
        _mix2_block,
        grid=grid,
        in_specs=[
            pl.BlockSpec((rb, fout), lambda i: (i, 0)),
            pl.BlockSpec((rb, fin), lambda i: (i, 0)),
            pl.BlockSpec((rb, fin), lambda i: (i + nb, 0)),
            pl.BlockSpec((fin, kk, fout), lambda i: (0, 0, 0)),
        ],
        out_specs=pl.BlockSpec((rb, fout), lambda i: (i, 0)),
        out_shape=jax.ShapeDtypeStruct((n, fout), jnp.float32),
    )(acc, q, q, weight)

    return out.reshape(b, v, fout)


# speedup vs baseline: 6.1654x; 1.0115x over previous
import functools

import jax
import jax.numpy as jnp
from jax import lax
from jax.experimental import pallas as pl
from jax.experimental.pallas import tpu as pltpu
from jax.experimental.pallas import tpu_sc as plsc

NC = 2
NS = 16
L = 16
C = 128


def _spmm_body(n_rows, t0, t1, x_hbm, col_hbm, row_hbm, val_hbm, out_hbm,
               col_v, row_v, val_v, rows_v, acc, sem_i, sem_g, sem_s):
    c = lax.axis_index("c")
    s = lax.axis_index("s")
    f = x_hbm.shape[1]
    nj = f // L
    zrows = rows_v.shape[1]

    stripe = -(-(n_rows // NS) // 8) * 8
    start = s * stripe
    nrows = jnp.minimum(stripe, n_rows - start)

    zero = jnp.zeros((L,), jnp.float32)

    def zb(i, _):
        for j in range(nj):
            rows_v[0, i, pl.ds(j * L, L)] = zero
        return 0

    nbig = nrows // zrows
    tail0 = start + nbig * zrows
    ntail = (nrows - nbig * zrows) // 8

    with jax.named_scope("zero_acc"):
        lax.fori_loop(0, zrows, zb, 0)

        def zcp(i, _):
            pltpu.sync_copy(rows_v.at[0], acc.at[pl.ds(start + i * zrows, zrows)])
            return 0

        lax.fori_loop(0, nbig, zcp, 0)

        def zcp8(i, _):
            pltpu.sync_copy(rows_v.at[0, pl.ds(0, 8)], acc.at[pl.ds(tail0 + i * 8, 8)])
            return 0

        lax.fori_loop(0, ntail, zcp8, 0)
        plsc.subcore_barrier()

    base = jnp.where(c == 0, s * t0, NS * t0 + s * t1) * C
    nt = jnp.where(c == 0, t0, t1)

    def start_idx(t):
        b = lax.rem(t, 3)
        off = base + t * C
        pltpu.make_async_copy(col_hbm.at[pl.ds(off, C)], col_v.at[b], sem_i.at[b]).start()
        pltpu.make_async_copy(row_hbm.at[pl.ds(off, C)], row_v.at[b], sem_i.at[b]).start()
        pltpu.make_async_copy(val_hbm.at[pl.ds(off, C)], val_v.at[b], sem_i.at[b]).start()

    def wait_idx(t):
        b = lax.rem(t, 3)
        pltpu.make_async_copy(col_hbm.at[pl.ds(base, C)], col_v.at[b], sem_i.at[b]).wait()
        pltpu.make_async_copy(row_hbm.at[pl.ds(base, C)], row_v.at[b], sem_i.at[b]).wait()
        pltpu.make_async_copy(val_hbm.at[pl.ds(base, C)], val_v.at[b], sem_i.at[b]).wait()

    def start_gather(t):
        b = lax.rem(t, 3)
        pltpu.make_async_copy(x_hbm.at[col_v.at[b]], rows_v.at[b], sem_g.at[b]).start()

    def wait_gather(t):
        b = lax.rem(t, 3)
        pltpu.make_async_copy(x_hbm.at[col_v.at[b]], rows_v.at[b], sem_g.at[b]).wait()

    def start_scatter(t):
        b = lax.rem(t, 3)
        pltpu.async_copy(rows_v.at[b], acc.at[row_v.at[b]], sem_s.at[b], add=True)

    def wait_scatter(t):
        b = lax.rem(t, 3)
        pltpu.make_async_copy(rows_v.at[b], acc.at[row_v.at[b]], sem_s.at[b]).wait()

    sco = jax.named_scope("edge_loop")
    sco.__enter__()
    start_idx(0)
    start_idx(1)
    wait_idx(0)
    start_gather(0)

    def chunk(t, _):
        b = lax.rem(t, 3)
        wait_gather(t)

        @pl.when(t + 1 < nt)
        def _():
            wait_idx(t + 1)
            start_gather(t + 1)

        @plsc.parallel_loop(0, C // L, 1, unroll=2)
        def scale(g):
            vg = val_v[b, pl.ds(g * L, L)]
            for l in range(L):
                e = g * L + l
                vv = vg[l]
                segs = [rows_v[b, e, pl.ds(j * L, L)] * vv for j in range(nj)]
                for j in range(nj):
                    rows_v[b, e, pl.ds(j * L, L)] = segs[j]

        @pl.when(t >= 1)
        def _():
            wait_scatter(t - 1)

        start_scatter(t)

        @pl.when(t + 2 < nt)
        def _():
            start_idx(t + 2)

        return 0

    lax.fori_loop(0, nt, chunk, 0)
    wait_scatter(nt - 1)
    sco.__exit__(None, None, None)

    with jax.named_scope("writeout"):
        plsc.subcore_barrier()
        cn = c * n_rows

        def wcp(i, _):
            o = start + i * zrows
            pltpu.sync_copy(acc.at[pl.ds(o, zrows)], rows_v.at[0])
            pltpu.sync_copy(rows_v.at[0], out_hbm.at[pl.ds(cn + o, zrows)])
            return 0

        lax.fori_loop(0, nbig, wcp, 0)

        def wcp8(i, _):
            o = tail0 + i * 8
            pltpu.sync_copy(acc.at[pl.ds(o, 8)], rows_v.at[0, pl.ds(0, 8)])
            pltpu.sync_copy(rows_v.at[0, pl.ds(0, 8)], out_hbm.at[pl.ds(cn + o, 8)])
            return 0

        lax.fori_loop(0, ntail, wcp8, 0)


def _spmm_sc(xmat, col, row, val, t0, t1):
    n, f = xmat.shape
    mesh = plsc.VectorSubcoreMesh(core_axis_name="c", subcore_axis_name="s")
    kern = functools.partial(
        pl.kernel,
        mesh=mesh,
        out_type=jax.ShapeDtypeStruct((NC * n, f), jnp.float32),
        scratch_types=[
            pltpu.VMEM((3, C), jnp.int32),
            pltpu.VMEM((3, C), jnp.int32),
            pltpu.VMEM((3, C), jnp.float32),
            pltpu.VMEM((3, C, f), jnp.float32),
            pltpu.VMEM_SHARED((n, f), jnp.float32),
            pltpu.SemaphoreType.DMA((3,)),
            pltpu.SemaphoreType.DMA((3,)),
            pltpu.SemaphoreType.DMA((3,)),
        ],
    )(functools.partial(_spmm_body, n, t0, t1))
    return kern(xmat, col, row, val)


def _mix1_block(x0_ref, pa_ref, pb_ref, w_ref, b_ref, x1_ref, acc_ref):
    x1 = pa_ref[...] + pb_ref[...]
    x1_ref[...] = x1
    w = w_ref[...]
    w02 = w[:, 0, :] - w[:, 2, :]
    w1 = w[:, 1, :]
    acc_ref[...] = (jnp.dot(x0_ref[...], w02, preferred_element_type=jnp.float32)
                    + jnp.dot(x1, w1, preferred_element_type=jnp.float32)
                    + b_ref[...])


def _mix2_block(acc_ref, qa_ref, qb_ref, w_ref, out_ref):
    w2 = w_ref[...][:, 2, :]
    q = qa_ref[...] + qb_ref[...]
    out_ref[...] = acc_ref[...] + jnp.dot(2.0 * q, w2,
                                          preferred_element_type=jnp.float32)


def kernel(x, edge_index, edge_values, weight, bias):
    b, v, fin = x.shape
    fin2, kk, fout = weight.shape
    n = b * v
    x0 = x.reshape(n, fin)

    row = edge_index[0].astype(jnp.int32)
    col = edge_index[1].astype(jnp.int32)
    e = row.shape[0]
    per_sub = NC * NS * C
    total = NC * (-(-e // per_sub))
    t0 = (2 * total) // 3
    t1 = total - t0
    e_pad = NS * C * total
    pad = e_pad - e
    row = jnp.pad(row, (0, pad))
    col = jnp.pad(col, (0, pad))
    val = jnp.pad(edge_values, (0, pad))

    p = _spmm_sc(x0, col, row, val, t0, t1)

    rb = 1000
    nb = n // rb
    grid = (nb,)
    bias2 = bias.reshape(1, fout)
    x1, acc = pl.pallas_call(
        _mix1_block,
        grid=grid,
        in_specs=[
            pl.BlockSpec((rb, fin), lambda i: (i, 0)),
            pl.BlockSpec((rb, fin), lambda i: (i, 0)),
            pl.BlockSpec((rb, fin), lambda i: (i + nb, 0)),
            pl.BlockSpec((fin, kk, fout), lambda i: (0, 0, 0)),
            pl.BlockSpec((1, fout), lambda i: (0, 0)),
        ],
        out_specs=[
            pl.BlockSpec((rb, fin), lambda i: (i, 0)),
            pl.BlockSpec((rb, fout), lambda i: (i, 0)),
        ],
        out_shape=[
            jax.ShapeDtypeStruct((n, fin), jnp.float32),
            jax.ShapeDtypeStruct((n, fout), jnp.float32),
        ],
    )(x0, p, p, weight, bias2)

    q = _spmm_sc(x1, col, row, val, t0, t1)

    out = pl.pallas_call(
        _mix2_block,
        grid=grid,
        in_specs=[
            pl.BlockSpec((rb, fout), lambda i: (i, 0)),
            pl.BlockSpec((rb, fin), lambda i: (i, 0)),
            pl.BlockSpec((rb, fin), lambda i: (i + nb, 0)),
            pl.BlockSpec((fin, kk, fout), lambda i: (0, 0, 0)),
        ],
        out_specs=pl.BlockSpec((rb, fout), lambda i: (i, 0)),
        out_shape=jax.ShapeDtypeStruct((n, fout), jnp.float32),
    )(acc, q, q, weight)

    return out.reshape(b, v, fout)


# speedup vs baseline: 6.8171x; 1.1057x over previous
import functools

import jax
import jax.numpy as jnp
from jax import lax
from jax.experimental import pallas as pl
from jax.experimental.pallas import tpu as pltpu
from jax.experimental.pallas import tpu_sc as plsc

NC = 2
NS = 16
L = 16
C = 128


def _spmm_body(n_rows, t0, t1, x_hbm, col_hbm, row_hbm, val_hbm, out_hbm,
               col_v, row_v, val_v, rows_v, acc, sem_i, sem_g, sem_s):
    c = lax.axis_index("c")
    s = lax.axis_index("s")
    f = x_hbm.shape[1]
    nj = f // L
    zrows = rows_v.shape[1]

    stripe = -(-(n_rows // NS) // 16) * 16
    start = s * stripe
    nrows = jnp.minimum(stripe, n_rows - start)

    zero = jnp.zeros((L,), jnp.float32)

    def zb(i, _):
        for j in range(nj):
            rows_v[0, i, pl.ds(j * L, L)] = zero
        return 0

    nbig = nrows // zrows
    tail0 = start + nbig * zrows
    ntail = (nrows - nbig * zrows) // 16

    with jax.named_scope("zero_acc"):
        lax.fori_loop(0, zrows, zb, 0)

        def zcp(i, _):
            pltpu.sync_copy(rows_v.at[0], acc.at[pl.ds(start + i * zrows, zrows)])
            return 0

        lax.fori_loop(0, nbig, zcp, 0)

        def zcp16(i, _):
            pltpu.sync_copy(rows_v.at[0, pl.ds(0, 16)],
                            acc.at[pl.ds(tail0 + i * 16, 16)])
            return 0

        lax.fori_loop(0, ntail, zcp16, 0)
        plsc.subcore_barrier()

    base = jnp.where(c == 0, s * t0, NS * t0 + s * t1) * C
    nt = jnp.where(c == 0, t0, t1)

    def start_idx(t):
        b = lax.rem(t, 3)
        off = base + t * C
        pltpu.make_async_copy(col_hbm.at[pl.ds(off, C)], col_v.at[b], sem_i.at[b]).start()
        pltpu.make_async_copy(row_hbm.at[pl.ds(off, C)], row_v.at[b], sem_i.at[b]).start()
        pltpu.make_async_copy(val_hbm.at[pl.ds(off, C)], val_v.at[b], sem_i.at[b]).start()

    def wait_idx(t):
        b = lax.rem(t, 3)
        pltpu.make_async_copy(col_hbm.at[pl.ds(base, C)], col_v.at[b], sem_i.at[b]).wait()
        pltpu.make_async_copy(row_hbm.at[pl.ds(base, C)], row_v.at[b], sem_i.at[b]).wait()
        pltpu.make_async_copy(val_hbm.at[pl.ds(base, C)], val_v.at[b], sem_i.at[b]).wait()

    def start_gather(t):
        b = lax.rem(t, 3)
        pltpu.make_async_copy(x_hbm.at[col_v.at[b]], rows_v.at[b], sem_g.at[b]).start()

    def wait_gather(t):
        b = lax.rem(t, 3)
        pltpu.make_async_copy(x_hbm.at[col_v.at[b]], rows_v.at[b], sem_g.at[b]).wait()

    def start_scatter(t):
        b = lax.rem(t, 3)
        pltpu.async_copy(rows_v.at[b], acc.at[row_v.at[b]], sem_s.at[b], add=True)

    def wait_scatter(t):
        b = lax.rem(t, 3)
        pltpu.make_async_copy(rows_v.at[b], acc.at[row_v.at[b]], sem_s.at[b]).wait()

    sco = jax.named_scope("edge_loop")
    sco.__enter__()
    start_idx(0)
    start_idx(1)
    wait_idx(0)
    start_gather(0)

    def chunk(t, _):
        b = lax.rem(t, 3)
        wait_gather(t)

        @pl.when(t + 1 < nt)
        def _():
            wait_idx(t + 1)
            start_gather(t + 1)

        @plsc.parallel_loop(0, C // L, 1, unroll=2)
        def scale(g):
            vg = val_v[b, pl.ds(g * L, L)]
            for l in range(L):
                e = g * L + l
                vv = vg[l]
                segs = [rows_v[b, e, pl.ds(j * L, L)] * vv for j in range(nj)]
                for j in range(nj):
                    rows_v[b, e, pl.ds(j * L, L)] = segs[j]

        @pl.when(t >= 1)
        def _():
            wait_scatter(t - 1)

        start_scatter(t)

        @pl.when(t + 2 < nt)
        def _():
            start_idx(t + 2)

        return 0

    lax.fori_loop(0, nt, chunk, 0)
    wait_scatter(nt - 1)
    sco.__exit__(None, None, None)

    def _rne16(u):
        return u + 0x7FFF + (lax.shift_right_logical(u, 16) & 1)

    def pack_rows(m):
        @plsc.parallel_loop(0, m // 2, 1, unroll=2)
        def _(i):
            for j in range(nj):
                a = rows_v[0, 2 * i, pl.ds(L * j, L)]
                bq = rows_v[0, 2 * i + 1, pl.ds(L * j, L)]
                ua = plsc.bitcast(a, jnp.int32)
                ub = plsc.bitcast(bq, jnp.int32)
                lo = lax.shift_right_logical(_rne16(ua), 16)
                hi = _rne16(ub) & jnp.int32(-65536)
                rows_v[1, i, pl.ds(L * j, L)] = plsc.bitcast(lo | hi, jnp.float32)

    with jax.named_scope("writeout"):
        plsc.subcore_barrier()

        start2 = s * (stripe // 2)

        def wcp(i, _):
            o = start + i * zrows
            pltpu.sync_copy(acc.at[pl.ds(o, zrows)], rows_v.at[0])
            pack_rows(zrows)
            pltpu.sync_copy(rows_v.at[1, pl.ds(0, zrows // 2)],
                            out_hbm.at[c, pl.ds(start2 + i * (zrows // 2),
                                                zrows // 2)])
            return 0

        lax.fori_loop(0, nbig, wcp, 0)
        tail2 = start2 + nbig * (zrows // 2)

        def wcp16(i, _):
            o = tail0 + i * 16
            pltpu.sync_copy(acc.at[pl.ds(o, 16)], rows_v.at[0, pl.ds(0, 16)])
            pack_rows(16)
            pltpu.sync_copy(rows_v.at[1, pl.ds(0, 8)],
                            out_hbm.at[c, pl.ds(tail2 + i * 8, 8)])
            return 0

        lax.fori_loop(0, ntail, wcp16, 0)


def _spmm_sc(xmat, col, row, val, t0, t1):
    n, f = xmat.shape
    mesh = plsc.VectorSubcoreMesh(core_axis_name="c", subcore_axis_name="s")
    kern = functools.partial(
        pl.kernel,
        mesh=mesh,
        compiler_params=pltpu.CompilerParams(needs_layout_passes=False),
        out_type=jax.ShapeDtypeStruct((NC, n // 2, f), jnp.float32),
        scratch_types=[
            pltpu.VMEM((3, C), jnp.int32),
            pltpu.VMEM((3, C), jnp.int32),
            pltpu.VMEM((3, C), jnp.float32),
            pltpu.VMEM((3, C, f), jnp.float32),
            pltpu.VMEM_SHARED((n, f), jnp.float32),
            pltpu.SemaphoreType.DMA((3,)),
            pltpu.SemaphoreType.DMA((3,)),
            pltpu.SemaphoreType.DMA((3,)),
        ],
    )(functools.partial(_spmm_body, n, t0, t1))
    return kern(xmat, col, row, val)


def _unpack_bf16(x):
    return pltpu.bitcast(x, jnp.bfloat16).astype(jnp.float32)


def _mix1_block(x0_ref, pa_ref, pb_ref, w02_ref, w1_ref, b_ref, x1_ref, acc_ref):
    x1 = _unpack_bf16(pa_ref[0]) + _unpack_bf16(pb_ref[0])
    x1_ref[...] = x1
    acc_ref[...] = (jnp.dot(x0_ref[...], w02_ref[...],
                            preferred_element_type=jnp.float32)
                    + jnp.dot(x1, w1_ref[...],
                              preferred_element_type=jnp.float32)
                    + b_ref[...])


def _mix2_block(acc_ref, qa_ref, qb_ref, w2_ref, out_ref):
    q = _unpack_bf16(qa_ref[0]) + _unpack_bf16(qb_ref[0])
    out_ref[...] = acc_ref[...] + jnp.dot(2.0 * q, w2_ref[...],
                                          preferred_element_type=jnp.float32)


def kernel(x, edge_index, edge_values, weight, bias):
    b, v, fin = x.shape
    fin2, kk, fout = weight.shape
    n = b * v
    x0 = x.reshape(n, fin)

    row = edge_index[0].astype(jnp.int32)
    col = edge_index[1].astype(jnp.int32)
    e = row.shape[0]
    per_sub = NC * NS * C
    total = NC * (-(-e // per_sub))
    t0 = (4 * total) // 5
    t1 = total - t0
    e_pad = NS * C * total
    pad = e_pad - e
    row = jnp.pad(row, (0, pad))
    col = jnp.pad(col, (0, pad))
    val = jnp.pad(edge_values, (0, pad))

    p = _spmm_sc(x0, col, row, val, t0, t1)

    w02 = weight[:, 0, :] - weight[:, 2, :]
    w1 = weight[:, 1, :]
    w2 = weight[:, 2, :]

    rb = 2000
    rbh = rb // 2
    nb = n // rb
    grid = (nb,)
    bias2 = bias.reshape(1, fout)
    x1, acc = pl.pallas_call(
        _mix1_block,
        grid=grid,
        in_specs=[
            pl.BlockSpec((rb, fin), lambda i: (i, 0)),
            pl.BlockSpec((1, rbh, fin), lambda i: (0, i, 0)),
            pl.BlockSpec((1, rbh, fin), lambda i: (1, i, 0)),
            pl.BlockSpec((fin, fout), lambda i: (0, 0)),
            pl.BlockSpec((fin, fout), lambda i: (0, 0)),
            pl.BlockSpec((1, fout), lambda i: (0, 0)),
        ],
        out_specs=[
            pl.BlockSpec((rb, fin), lambda i: (i, 0)),
            pl.BlockSpec((rb, fout), lambda i: (i, 0)),
        ],
        out_shape=[
            jax.ShapeDtypeStruct((n, fin), jnp.float32),
            jax.ShapeDtypeStruct((n, fout), jnp.float32),
        ],
    )(x0, p, p, w02, w1, bias2)

    q = _spmm_sc(x1, col, row, val, t0, t1)

    out = pl.pallas_call(
        _mix2_block,
        grid=grid,
        in_specs=[
            pl.BlockSpec((rb, fout), lambda i: (i, 0)),
            pl.BlockSpec((1, rbh, fin), lambda i: (0, i, 0)),
            pl.BlockSpec((1, rbh, fin), lambda i: (1, i, 0)),
            pl.BlockSpec((fin, fout), lambda i: (0, 0)),
        ],
        out_specs=pl.BlockSpec((rb, fout), lambda i: (i, 0)),
        out_shape=jax.ShapeDtypeStruct((n, fout), jnp.float32),
    )(acc, q, q, w2)

    return out.reshape(b, v, fout)
